# baseline (speedup 1.0000x reference)
import jax
import jax.numpy as jnp
from jax.experimental import pallas as pl
from jax.experimental.pallas import tpu as pltpu


def _pool_kernel(x0_ref, x1_ref, x2_ref, w_ref, out_ref,
                 m_ref, z_ref, s_ref):
    i = pl.program_id(0)

    @pl.when(i == 0)
    def _init():
        m_ref[...] = jnp.full_like(m_ref, -jnp.inf)
        z_ref[...] = jnp.zeros_like(z_ref)
        s_ref[...] = jnp.zeros_like(s_ref)

    m = m_ref[...]
    z = z_ref[...]
    s = s_ref[...]
    w = w_ref[...]
    for x_ref in (x0_ref, x1_ref, x2_ref):
        x = x_ref[...]
        g = jnp.dot(x, w, preferred_element_type=jnp.float32)
        tile_m = jnp.max(g, axis=0, keepdims=True)
        new_m = jnp.maximum(m, tile_m)
        alpha = jnp.exp(m - new_m)
        e = jnp.exp(g - new_m)
        ones_row = jnp.ones((1, x.shape[0]), dtype=jnp.float32)
        z = z * alpha + jnp.dot(ones_row, e, preferred_element_type=jnp.float32)
        s = s * alpha + jnp.dot(ones_row, x * e, preferred_element_type=jnp.float32)
        m = new_m
    m_ref[...] = m
    z_ref[...] = z
    s_ref[...] = s

    @pl.when(i == pl.num_programs(0) - 1)
    def _fin():
        out_ref[...] = jnp.sum(s / z, axis=(0, 1), keepdims=True)


def kernel(feat_ntype0, feat_ntype1, feat_ntype2, W_gate, b_gate):
    n0, t = feat_ntype0.shape
    n1 = feat_ntype1.shape[0]
    n2 = feat_ntype2.shape[0]

    grid_n = None
    for g in range(1, min(n0, n1, n2) // 8 + 1):
        if n0 % g == 0 and n1 % g == 0 and n2 % g == 0 \
                and (n0 // g) % 8 == 0 and (n1 // g) % 8 == 0 and (n2 // g) % 8 == 0:
            grid_n = g
            if max(n0, n1, n2) // g <= 4096:
                break
    t0, t1, t2 = n0 // grid_n, n1 // grid_n, n2 // grid_n

    del b_gate
    out = pl.pallas_call(
        _pool_kernel,
        grid=(grid_n,),
        in_specs=[
            pl.BlockSpec((t0, t), lambda i: (i, 0)),
            pl.BlockSpec((t1, t), lambda i: (i, 0)),
            pl.BlockSpec((t2, t), lambda i: (i, 0)),
            pl.BlockSpec((t, t), lambda i: (0, 0)),
        ],
        out_specs=pl.BlockSpec((1, 1), lambda i: (0, 0)),
        out_shape=jax.ShapeDtypeStruct((1, 1), jnp.float32),
        scratch_shapes=[pltpu.VMEM((1, t), jnp.float32)] * 3,
    )(feat_ntype0, feat_ntype1, feat_ntype2, W_gate)
    return out.reshape(1)


# speedup vs baseline: 1.0912x; 1.0912x over previous
import jax
import jax.numpy as jnp
from jax.experimental import pallas as pl
from jax.experimental.pallas import tpu as pltpu


def _pool_kernel(x0_ref, x1_ref, x2_ref, w_ref, out_ref,
                 m_ref, z_ref, s_ref):
    i = pl.program_id(0)

    @pl.when(i == 0)
    def _init():
        m_ref[...] = jnp.full_like(m_ref, -jnp.inf)
        z_ref[...] = jnp.zeros_like(z_ref)
        s_ref[...] = jnp.zeros_like(s_ref)

    m = m_ref[...]
    z = z_ref[...]
    s = s_ref[...]
    w = w_ref[...] * jnp.float32(1.4426950408889634)
    for x_ref in (x0_ref, x1_ref, x2_ref):
        x = x_ref[...]
        g = jnp.dot(x, w, preferred_element_type=jnp.float32)
        tile_m = jnp.max(g, axis=0, keepdims=True)
        new_m = jnp.maximum(m, tile_m)
        alpha = jnp.exp2(m - new_m)
        e = jnp.exp2(g - new_m)
        z = z * alpha + jnp.sum(e, axis=0, keepdims=True)
        s = s * alpha + jnp.sum(x * e, axis=0, keepdims=True)
        m = new_m
    m_ref[...] = m
    z_ref[...] = z
    s_ref[...] = s

    @pl.when(i == pl.num_programs(0) - 1)
    def _fin():
        out_ref[...] = jnp.sum(s / z, axis=(0, 1), keepdims=True)


def kernel(feat_ntype0, feat_ntype1, feat_ntype2, W_gate, b_gate):
    n0, t = feat_ntype0.shape
    n1 = feat_ntype1.shape[0]
    n2 = feat_ntype2.shape[0]

    grid_n = None
    for g in range(1, min(n0, n1, n2) // 8 + 1):
        if n0 % g == 0 and n1 % g == 0 and n2 % g == 0 \
                and (n0 // g) % 8 == 0 and (n1 // g) % 8 == 0 and (n2 // g) % 8 == 0:
            grid_n = g
            if max(n0, n1, n2) // g <= 4096:
                break
    t0, t1, t2 = n0 // grid_n, n1 // grid_n, n2 // grid_n

    del b_gate
    out = pl.pallas_call(
        _pool_kernel,
        grid=(grid_n,),
        in_specs=[
            pl.BlockSpec((t0, t), lambda i: (i, 0)),
            pl.BlockSpec((t1, t), lambda i: (i, 0)),
            pl.BlockSpec((t2, t), lambda i: (i, 0)),
            pl.BlockSpec((t, t), lambda i: (0, 0)),
        ],
        out_specs=pl.BlockSpec((1, 1), lambda i: (0, 0)),
        out_shape=jax.ShapeDtypeStruct((1, 1), jnp.float32),
        scratch_shapes=[pltpu.VMEM((1, t), jnp.float32)] * 3,
    )(feat_ntype0, feat_ntype1, feat_ntype2, W_gate)
    return out.reshape(1)
